# baseline (speedup 1.0000x reference)
import functools
import math

import jax
import jax.numpy as jnp
from jax import lax
from jax.experimental import pallas as pl
from jax.experimental.pallas import tpu as pltpu
from jax.experimental.pallas import tpu_sc as plsc

EMB = 64
SCALE = math.sqrt(EMB)
GSZ = 128
CHUNK = 512
NSUB = CHUNK // GSZ


def _make_sc_kernel(B, V):
    info = plsc.get_sparse_core_info()
    NC, NS = info.num_cores, info.num_subcores
    NW = NC * NS
    per_w = B // NW
    assert B % NW == 0 and per_w % CHUNK == 0
    n_blk = per_w // CHUNK
    rows_per_w = per_w // GSZ

    mesh = plsc.VectorSubcoreMesh(core_axis_name="c", subcore_axis_name="s")

    @functools.partial(
        pl.kernel,
        mesh=mesh,
        compiler_params=pltpu.CompilerParams(use_tc_tiling_on_sc=False),
        out_type=jax.ShapeDtypeStruct((B, EMB), jnp.float32),
        scratch_types=[
            pltpu.VMEM((rows_per_w, GSZ), jnp.int32),
            pltpu.VMEM((CHUNK, EMB), jnp.float32),
            pltpu.SemaphoreType.DMA,
        ],
    )
    def k(x_hbm, table_hbm, out_hbm, idx_v, rows_v, sem):
        wid = lax.axis_index("s") * NC + lax.axis_index("c")
        base = wid * per_w
        pltpu.sync_copy(x_hbm.at[wid], idx_v)

        def body(blk, _):
            for sub in range(NSUB):
                pltpu.async_copy(
                    table_hbm.at[idx_v.at[blk * NSUB + sub]],
                    rows_v.at[pl.ds(sub * GSZ, GSZ)],
                    sem,
                ).wait()

            def srow(i, _):
                for j in range(EMB // 16):
                    rows_v[i, pl.ds(j * 16, 16)] = (
                        rows_v[i, pl.ds(j * 16, 16)] * SCALE
                    )
                return 0

            lax.fori_loop(0, CHUNK, srow, 0)
            pltpu.sync_copy(
                rows_v, out_hbm.at[pl.ds(base + blk * CHUNK, CHUNK)]
            )
            return 0

        lax.fori_loop(0, n_blk, body, 0)

    def run(x, table):
        xr = x.reshape(NW, rows_per_w, GSZ)
        return k(xr, table)

    return run


def kernel(x, table):
    batch, hist = x.shape
    B = batch * hist
    out = _make_sc_kernel(B, table.shape[0])(x.astype(jnp.int32), table)
    return out.reshape(batch, hist, EMB)


# speedup vs baseline: 1.1857x; 1.1857x over previous
import functools
import math

import jax
import jax.numpy as jnp
from jax import lax
from jax.experimental import pallas as pl
from jax.experimental.pallas import tpu as pltpu
from jax.experimental.pallas import tpu_sc as plsc

EMB = 64
SCALE = math.sqrt(EMB)
GSZ = 128
CHUNK = 512
NSUB = CHUNK // GSZ


def _make_sc_kernel(B, V):
    info = plsc.get_sparse_core_info()
    NC, NS = info.num_cores, info.num_subcores
    NW = NC * NS
    per_w = B // NW
    assert B % NW == 0 and per_w % (2 * CHUNK) == 0
    n_blk = per_w // CHUNK
    n_pair = n_blk // 2
    rows_per_w = per_w // GSZ

    mesh = plsc.VectorSubcoreMesh(core_axis_name="c", subcore_axis_name="s")

    @functools.partial(
        pl.kernel,
        mesh=mesh,
        compiler_params=pltpu.CompilerParams(use_tc_tiling_on_sc=False),
        out_type=jax.ShapeDtypeStruct((B, EMB), jnp.float32),
        scratch_types=[
            pltpu.VMEM((rows_per_w, GSZ), jnp.int32),
            pltpu.VMEM((CHUNK, EMB), jnp.float32),
            pltpu.VMEM((CHUNK, EMB), jnp.float32),
            pltpu.SemaphoreType.DMA,
            pltpu.SemaphoreType.DMA,
            pltpu.SemaphoreType.DMA,
            pltpu.SemaphoreType.DMA,
        ],
    )
    def k(x_hbm, table_hbm, out_hbm, idx_v, rows0, rows1, g0, g1, s0, s1):
        wid = lax.axis_index("s") * NC + lax.axis_index("c")
        base = wid * per_w
        pltpu.sync_copy(x_hbm.at[wid], idx_v)

        rows = (rows0, rows1)
        gsem = (g0, g1)
        ssem = (s0, s1)
        dummy = out_hbm.at[pl.ds(0, CHUNK)]

        def fire(blk, buf):
            for sub in range(NSUB):
                pltpu.async_copy(
                    table_hbm.at[idx_v.at[blk * NSUB + sub]],
                    rows[buf].at[pl.ds(sub * GSZ, GSZ)],
                    gsem[buf],
                )

        def scale(buf):
            def srow(i, _):
                for j in range(EMB // 16):
                    rows[buf][i, pl.ds(j * 16, 16)] = (
                        rows[buf][i, pl.ds(j * 16, 16)] * SCALE
                    )
                return 0

            lax.fori_loop(0, CHUNK, srow, 0, unroll=8)

        fire(0, 0)

        def body(kk, _):
            for b in range(2):
                cur = 2 * kk + b
                o = 1 - b
                if b == 0:

                    @pl.when(kk >= 1)
                    def _():
                        pltpu.make_async_copy(rows[o], dummy, ssem[o]).wait()

                    fire(cur + 1, o)
                else:
                    pltpu.make_async_copy(rows[o], dummy, ssem[o]).wait()

                    @pl.when(kk < n_pair - 1)
                    def _():
                        fire(cur + 1, o)

                pltpu.make_async_copy(dummy, rows[b], gsem[b]).wait()
                scale(b)
                pltpu.async_copy(
                    rows[b],
                    out_hbm.at[pl.ds(base + cur * CHUNK, CHUNK)],
                    ssem[b],
                )
            return 0

        lax.fori_loop(0, n_pair, body, 0)
        pltpu.make_async_copy(rows1, dummy, ssem[1]).wait()

    def run(x, table):
        xr = x.reshape(NW, rows_per_w, GSZ)
        return k(xr, table)

    return run


def kernel(x, table):
    batch, hist = x.shape
    B = batch * hist
    out = _make_sc_kernel(B, table.shape[0])(x.astype(jnp.int32), table)
    return out.reshape(batch, hist, EMB)


# speedup vs baseline: 1.1876x; 1.0016x over previous
import functools
import math

import jax
import jax.numpy as jnp
from jax import lax
from jax.experimental import pallas as pl
from jax.experimental.pallas import tpu as pltpu
from jax.experimental.pallas import tpu_sc as plsc

EMB = 64
SCALE = math.sqrt(EMB)
GSZ = 512
CHUNK = 512
NSUB = CHUNK // GSZ


def _make_sc_kernel(B, V):
    info = plsc.get_sparse_core_info()
    NC, NS = info.num_cores, info.num_subcores
    NW = NC * NS
    per_w = B // NW
    assert B % NW == 0 and per_w % (2 * CHUNK) == 0
    n_blk = per_w // CHUNK
    n_pair = n_blk // 2
    rows_per_w = per_w // GSZ

    mesh = plsc.VectorSubcoreMesh(core_axis_name="c", subcore_axis_name="s")

    @functools.partial(
        pl.kernel,
        mesh=mesh,
        compiler_params=pltpu.CompilerParams(use_tc_tiling_on_sc=False),
        out_type=jax.ShapeDtypeStruct((B, EMB), jnp.float32),
        scratch_types=[
            pltpu.VMEM((rows_per_w, GSZ), jnp.int32),
            pltpu.VMEM((CHUNK, EMB), jnp.float32),
            pltpu.VMEM((CHUNK, EMB), jnp.float32),
            pltpu.SemaphoreType.DMA,
            pltpu.SemaphoreType.DMA,
            pltpu.SemaphoreType.DMA,
            pltpu.SemaphoreType.DMA,
        ],
    )
    def k(x_hbm, table_hbm, out_hbm, idx_v, rows0, rows1, g0, g1, s0, s1):
        wid = lax.axis_index("s") * NC + lax.axis_index("c")
        base = wid * per_w
        pltpu.sync_copy(x_hbm.at[wid], idx_v)

        rows = (rows0, rows1)
        gsem = (g0, g1)
        ssem = (s0, s1)
        dummy = out_hbm.at[pl.ds(0, CHUNK)]

        def fire(blk, buf):
            for sub in range(NSUB):
                pltpu.async_copy(
                    table_hbm.at[idx_v.at[blk * NSUB + sub]],
                    rows[buf].at[pl.ds(sub * GSZ, GSZ)],
                    gsem[buf],
                )

        def scale(buf):
            def srow(i, _):
                for j in range(EMB // 16):
                    rows[buf][i, pl.ds(j * 16, 16)] = (
                        rows[buf][i, pl.ds(j * 16, 16)] * SCALE
                    )
                return 0

            lax.fori_loop(0, CHUNK, srow, 0, unroll=8)

        fire(0, 0)

        def body(kk, _):
            for b in range(2):
                cur = 2 * kk + b
                o = 1 - b
                if b == 0:

                    @pl.when(kk >= 1)
                    def _():
                        pltpu.make_async_copy(rows[o], dummy, ssem[o]).wait()

                    fire(cur + 1, o)
                else:
                    pltpu.make_async_copy(rows[o], dummy, ssem[o]).wait()

                    @pl.when(kk < n_pair - 1)
                    def _():
                        fire(cur + 1, o)

                pltpu.make_async_copy(dummy, rows[b], gsem[b]).wait()
                scale(b)
                pltpu.async_copy(
                    rows[b],
                    out_hbm.at[pl.ds(base + cur * CHUNK, CHUNK)],
                    ssem[b],
                )
            return 0

        lax.fori_loop(0, n_pair, body, 0)
        pltpu.make_async_copy(rows1, dummy, ssem[1]).wait()

    def run(x, table):
        xr = x.reshape(NW, rows_per_w, GSZ)
        return k(xr, table)

    return run


def kernel(x, table):
    batch, hist = x.shape
    B = batch * hist
    out = _make_sc_kernel(B, table.shape[0])(x.astype(jnp.int32), table)
    return out.reshape(batch, hist, EMB)
